# Initial kernel scaffold; baseline (speedup 1.0000x reference)
import jax
import jax.numpy as jnp
from jax.experimental import pallas as pl


def kernel(pos, edge_index, enc_W1, enc_b1, enc_W2, enc_b2, enc_W3, enc_b3, enc_W4, enc_b4, g1_W, g1_b, gh1_W, gh1_b, gh2_W, gh2_b, g2_W, g2_b, dec_W1, dec_b1, dec_W2, dec_b2, dec_W3, dec_b3, dec_W4, dec_b4):
    raise NotImplementedError("write your pallas kernel here")


# speedup vs baseline: 5.4400x; 5.4400x over previous
import functools

import numpy as np
import jax
import jax.numpy as jnp
from jax import lax
from jax.experimental import pallas as pl
from jax.experimental.pallas import tpu as pltpu
from jax.experimental.pallas import tpu_sc as plsc

N = 50000
NP = 50048
E = 800000
BOX = 1.0
NUM_CENTERS = 40
GAMMA = 40.0
_CENTERS = np.linspace(0.0, 1.0, NUM_CENTERS).astype(np.float32)

NC = 2
NS = 16
CHUNK = 128
NCHUNKS = E // CHUNK

_MESH = dict(core_axis_name="c", subcore_axis_name="s", num_cores=NC,
             num_subcores=NS)

ROWS_PER_TILE = NP // NS


def _tile_chunk_range(w, nworkers):
    per = NCHUNKS // nworkers
    extra = NCHUNKS - per * nworkers
    c0 = w * per + jnp.minimum(w, extra)
    nch = per + jnp.where(w < extra, 1, 0)
    return c0, nch


def _pre_body(pos16, srcA, dstA, ones_h, zdeg_h,
              spos_out, dpos_out, dego_out, degi_out,
              sidx, didx, srows, drows, ones_v,
              degosh, degish,
              semi, semg, semo, NB=4):
    cid = lax.axis_index("c")
    sid = lax.axis_index("s")
    w = cid * NS + sid
    c0, nch = _tile_chunk_range(w, NC * NS)

    r0 = sid * ROWS_PER_TILE
    pltpu.sync_copy(zdeg_h, degosh.at[pl.ds(r0, ROWS_PER_TILE)])
    pltpu.sync_copy(zdeg_h, degish.at[pl.ds(r0, ROWS_PER_TILE)])
    pltpu.sync_copy(ones_h, ones_v)
    plsc.subcore_barrier()

    def quad(q, carry):
        del carry
        idescs = []
        for p in range(NB):
            t = q * NB + p
            tcl = jnp.minimum(t, nch - 1)
            off = (c0 + tcl) * CHUNK
            d1 = pltpu.async_copy(srcA.at[pl.ds(off, CHUNK)], sidx[p], semi[p])
            d2d = pltpu.async_copy(dstA.at[pl.ds(off, CHUNK)], didx[p], semi[p])
            idescs.append((d1, d2d, off, t))
        gdescs = []
        for p in range(NB):
            idescs[p][0].wait()
            idescs[p][1].wait()
            g1 = pltpu.async_copy(pos16.at[sidx[p]], srows[p], semg[p])
            g2 = pltpu.async_copy(pos16.at[didx[p]], drows[p], semg[p])
            gdescs.append((g1, g2))
        odescs = []
        for p in range(NB):
            gdescs[p][0].wait()
            gdescs[p][1].wait()
            _, _, off, t = idescs[p]
            odescs.append(pltpu.async_copy(
                srows[p], spos_out.at[pl.ds(off, CHUNK)], semo[p]))
            odescs.append(pltpu.async_copy(
                drows[p], dpos_out.at[pl.ds(off, CHUNK)], semo[p]))

            @pl.when(t < nch)
            def _(p=p):
                pltpu.sync_copy(ones_v, degosh.at[sidx[p]], add=True)
                pltpu.sync_copy(ones_v, degish.at[didx[p]], add=True)
        for d in odescs:
            d.wait()
        return 0

    nq = (NCHUNKS // (NC * NS) + 1 + NB - 1) // NB
    lax.fori_loop(0, nq, quad, 0)

    plsc.subcore_barrier()
    pltpu.sync_copy(degosh.at[pl.ds(r0, ROWS_PER_TILE)],
                    dego_out.at[cid, pl.ds(r0, ROWS_PER_TILE)])
    pltpu.sync_copy(degish.at[pl.ds(r0, ROWS_PER_TILE)],
                    degi_out.at[cid, pl.ds(r0, ROWS_PER_TILE)])


def _sc_pre(pos16, srcA, dstA):
    ones_h = jnp.ones((CHUNK, 16), jnp.float32)
    zdeg_h = jnp.zeros((ROWS_PER_TILE, 16), jnp.float32)
    NB = 4
    scratch = (
        [pltpu.VMEM((CHUNK,), jnp.int32) for _ in range(NB)]
        + [pltpu.VMEM((CHUNK,), jnp.int32) for _ in range(NB)]
        + [pltpu.VMEM((CHUNK, 16), jnp.float32) for _ in range(NB)]
        + [pltpu.VMEM((CHUNK, 16), jnp.float32) for _ in range(NB)]
        + [pltpu.VMEM((CHUNK, 16), jnp.float32)]
        + [pltpu.VMEM_SHARED((NP, 16), jnp.float32)]
        + [pltpu.VMEM_SHARED((NP, 16), jnp.float32)]
        + [pltpu.SemaphoreType.DMA for _ in range(NB)]
        + [pltpu.SemaphoreType.DMA for _ in range(NB)]
        + [pltpu.SemaphoreType.DMA for _ in range(NB)]
    )

    def body(pos4_r, srcA_r, dstA_r, ones_r, zdeg_r,
             spos_r, dpos_r, dego_r, degi_r, *rest):
        sidx = list(rest[0:NB])
        didx = list(rest[NB:2 * NB])
        srows = list(rest[2 * NB:3 * NB])
        drows = list(rest[3 * NB:4 * NB])
        ones_v = rest[4 * NB]
        degosh = rest[4 * NB + 1]
        degish = rest[4 * NB + 2]
        semi = list(rest[4 * NB + 3:5 * NB + 3])
        semg = list(rest[5 * NB + 3:6 * NB + 3])
        semo = list(rest[6 * NB + 3:7 * NB + 3])
        _pre_body(pos4_r, srcA_r, dstA_r, ones_r, zdeg_r,
                  spos_r, dpos_r, dego_r, degi_r,
                  sidx, didx, srows, drows, ones_v,
                  degosh, degish, semi, semg, semo, NB=NB)

    f = pl.kernel(
        body,
        out_type=(jax.ShapeDtypeStruct((E, 16), jnp.float32),
                  jax.ShapeDtypeStruct((E, 16), jnp.float32),
                  jax.ShapeDtypeStruct((NC, NP, 16), jnp.float32),
                  jax.ShapeDtypeStruct((NC, NP, 16), jnp.float32)),
        mesh=plsc.VectorSubcoreMesh(**_MESH),
        scratch_types=scratch,
        compiler_params=pltpu.CompilerParams(use_tc_tiling_on_sc=False),
    )
    return f(pos16, srcA, dstA, ones_h, zdeg_h)


def _agg_body(hw2n, src2, dstA, zrows_h,
              agg_out,
              sidx, didx, rows, aggsh, semi, semg, NB=4):
    cid = lax.axis_index("c")
    sid = lax.axis_index("s")
    c0, nch = _tile_chunk_range(sid, NS)

    r0 = sid * ROWS_PER_TILE
    pltpu.sync_copy(zrows_h, aggsh.at[pl.ds(r0, ROWS_PER_TILE)])
    plsc.subcore_barrier()

    src_base = cid * E

    def quad(q, carry):
        del carry
        idescs = []
        for p in range(NB):
            t = q * NB + p
            tcl = jnp.minimum(t, nch - 1)
            off = (c0 + tcl) * CHUNK
            d1 = pltpu.async_copy(src2.at[pl.ds(src_base + off, CHUNK)],
                                  sidx[p], semi[p])
            d2d = pltpu.async_copy(dstA.at[pl.ds(off, CHUNK)], didx[p], semi[p])
            idescs.append((d1, d2d, t))
        gdescs = []
        for p in range(NB):
            idescs[p][0].wait()
            idescs[p][1].wait()
            gdescs.append(pltpu.async_copy(hw2n.at[sidx[p]], rows[p], semg[p]))
        for p in range(NB):
            gdescs[p].wait()
            t = idescs[p][2]

            @pl.when(t < nch)
            def _(p=p):
                pltpu.sync_copy(rows[p], aggsh.at[didx[p]], add=True)
        return 0

    nq = (NCHUNKS // NS + 1 + NB - 1) // NB
    lax.fori_loop(0, nq, quad, 0)

    plsc.subcore_barrier()
    pltpu.sync_copy(aggsh.at[pl.ds(r0, ROWS_PER_TILE)],
                    agg_out.at[cid, pl.ds(r0, ROWS_PER_TILE)])


def _sc_agg(hw2n, src2, dstA):
    zrows_h = jnp.zeros((ROWS_PER_TILE, 32), jnp.float32)
    NB = 4
    scratch = (
        [pltpu.VMEM((CHUNK,), jnp.int32) for _ in range(NB)]
        + [pltpu.VMEM((CHUNK,), jnp.int32) for _ in range(NB)]
        + [pltpu.VMEM((CHUNK, 32), jnp.float32) for _ in range(NB)]
        + [pltpu.VMEM_SHARED((NP, 32), jnp.float32)]
        + [pltpu.SemaphoreType.DMA for _ in range(NB)]
        + [pltpu.SemaphoreType.DMA for _ in range(NB)]
    )

    def body(hw_r, src2_r, dstA_r, z_r, agg_r, *rest):
        sidx = list(rest[0:NB])
        didx = list(rest[NB:2 * NB])
        rows = list(rest[2 * NB:3 * NB])
        aggsh = rest[3 * NB]
        semi = list(rest[3 * NB + 1:4 * NB + 1])
        semg = list(rest[4 * NB + 1:5 * NB + 1])
        _agg_body(hw_r, src2_r, dstA_r, z_r, agg_r,
                  sidx, didx, rows, aggsh, semi, semg, NB=NB)

    f = pl.kernel(
        body,
        out_type=jax.ShapeDtypeStruct((NC, NP, 32), jnp.float32),
        mesh=plsc.VectorSubcoreMesh(**_MESH),
        scratch_types=scratch,
        compiler_params=pltpu.CompilerParams(use_tc_tiling_on_sc=False),
    )
    return f(hw2n, src2, dstA, zrows_h)


ROWB = 2176
NGRID = NP // ROWB


def _leaky(x):
    return jnp.maximum(x, 0.2 * x)


def _full_spec(shape):
    nd = len(shape)
    return pl.BlockSpec(shape, lambda i, _nd=nd: (0,) * _nd)


def _tc_encode(pos, dego, degi, Ws, bs, g1_W):
    def body(pos_r, dego_r, degi_r, w1, b1, w2, b2, w3, b3, w4, b4, g1w,
             hw_r, ni_r, no_r):
        deg_o = dego_r[0, :, 0:1] + dego_r[1, :, 0:1]
        deg_i = degi_r[0, :, 0:1] + degi_r[1, :, 0:1]
        no = lax.rsqrt(jnp.maximum(deg_o, 1.0))
        ni = lax.rsqrt(jnp.maximum(deg_i, 1.0))
        h = _leaky(pos_r[...] @ w1[...] + b1[...])
        h = _leaky(h @ w2[...] + b2[...])
        h = _leaky(h @ w3[...] + b3[...])
        h = h @ w4[...] + b4[...]
        hw = (h * no) @ g1w[...]
        hw_r[0] = hw[:, :32]
        hw_r[1] = hw[:, 32:]
        ni_r[...] = ni
        no_r[...] = no

    in_specs = [
        pl.BlockSpec((ROWB, 3), lambda i: (i, 0)),
        pl.BlockSpec((NC, ROWB, 16), lambda i: (0, i, 0)),
        pl.BlockSpec((NC, ROWB, 16), lambda i: (0, i, 0)),
    ]
    args = [pos, dego, degi]
    for W, b in zip(Ws, bs):
        in_specs += [_full_spec(W.shape), _full_spec(b.shape)]
        args += [W, b]
    in_specs.append(_full_spec(g1_W.shape))
    args.append(g1_W)
    out_specs = (
        pl.BlockSpec((NC, ROWB, 32), lambda i: (0, i, 0)),
        pl.BlockSpec((ROWB, 1), lambda i: (i, 0)),
        pl.BlockSpec((ROWB, 1), lambda i: (i, 0)),
    )
    return pl.pallas_call(
        body,
        grid=(NGRID,),
        in_specs=in_specs,
        out_specs=out_specs,
        out_shape=(jax.ShapeDtypeStruct((NC, NP, 32), jnp.float32),
                   jax.ShapeDtypeStruct((NP, 1), jnp.float32),
                   jax.ShapeDtypeStruct((NP, 1), jnp.float32)),
    )(*args)


def _tc_mid(aggp, ni, no, b_prev, W_next, act):
    def body(agg_r, ni_r, no_r, b_r, w_r, hw_r):
        agg = jnp.concatenate([agg_r[0], agg_r[1]], axis=1)
        h = agg * ni_r[...] + b_r[...]
        if act is not None:
            h = act(h)
        hw = (h * no_r[...]) @ w_r[...]
        hw_r[0] = hw[:, :32]
        hw_r[1] = hw[:, 32:]

    return pl.pallas_call(
        body,
        grid=(NGRID,),
        in_specs=[
            pl.BlockSpec((NC, ROWB, 32), lambda i: (0, i, 0)),
            pl.BlockSpec((ROWB, 1), lambda i: (i, 0)),
            pl.BlockSpec((ROWB, 1), lambda i: (i, 0)),
            _full_spec(b_prev.shape),
            _full_spec(W_next.shape),
        ],
        out_specs=pl.BlockSpec((NC, ROWB, 32), lambda i: (0, i, 0)),
        out_shape=jax.ShapeDtypeStruct((NC, NP, 32), jnp.float32),
    )(aggp, ni, no, b_prev, W_next)


def _tc_final(aggp, ni, b_prev, Ws, bs):
    def body(agg_r, ni_r, b_r, w1, b1, w2, b2, w3, b3, w4, b4, out_r):
        agg = jnp.concatenate([agg_r[0], agg_r[1]], axis=1)
        h = agg * ni_r[...] + b_r[...]
        h = _leaky(h @ w1[...] + b1[...])
        h = _leaky(h @ w2[...] + b2[...])
        h = _leaky(h @ w3[...] + b3[...])
        out_r[...] = h @ w4[...] + b4[...]

    in_specs = [
        pl.BlockSpec((NC, ROWB, 32), lambda i: (0, i, 0)),
        pl.BlockSpec((ROWB, 1), lambda i: (i, 0)),
        _full_spec(b_prev.shape),
    ]
    args = [aggp, ni, b_prev]
    for W, b in zip(Ws, bs):
        in_specs += [_full_spec(W.shape), _full_spec(b.shape)]
        args += [W, b]
    return pl.pallas_call(
        body,
        grid=(NGRID,),
        in_specs=in_specs,
        out_specs=pl.BlockSpec((ROWB, 3), lambda i: (i, 0)),
        out_shape=jax.ShapeDtypeStruct((NP, 3), jnp.float32),
    )(*args)


EBLK = 4000
EGRID = E // EBLK


def _tc_edge_feat(spos, dpos, centers):
    def body(s_r, d_r, c_r, ef_r):
        sp = s_r[...]
        dp = d_r[...]
        d2 = None
        for comp in range(3):
            r = (dp[:, comp:comp + 1] - sp[:, comp:comp + 1]) + 0.5
            fl = jnp.floor(r)
            wv = r - fl - 0.5
            d2 = wv * wv if d2 is None else d2 + wv * wv
        dist = jnp.sqrt(d2)
        ef_r[...] = jnp.exp(-GAMMA * (dist - c_r[...]) ** 2)

    return pl.pallas_call(
        body,
        grid=(EGRID,),
        in_specs=[
            pl.BlockSpec((EBLK, 16), lambda i: (i, 0)),
            pl.BlockSpec((EBLK, 16), lambda i: (i, 0)),
            _full_spec((1, NUM_CENTERS)),
        ],
        out_specs=pl.BlockSpec((EBLK, NUM_CENTERS), lambda i: (i, 0)),
        out_shape=jax.ShapeDtypeStruct((E, NUM_CENTERS), jnp.float32),
    )(spos, dpos, centers)


def kernel(pos, edge_index, enc_W1, enc_b1, enc_W2, enc_b2, enc_W3, enc_b3,
           enc_W4, enc_b4, g1_W, g1_b, gh1_W, gh1_b, gh2_W, gh2_b, g2_W, g2_b,
           dec_W1, dec_b1, dec_W2, dec_b2, dec_W3, dec_b3, dec_W4, dec_b4):
    src = edge_index[0]
    dst = edge_index[1]
    pos16 = jnp.pad(pos, ((0, NP - N), (0, 13)))
    posP = jnp.pad(pos, ((0, NP - N), (0, 0)))
    src2 = jnp.concatenate([src, src + NP])

    spos, dpos, dego, degi = _sc_pre(pos16, src, dst)

    centers = jnp.asarray(_CENTERS).reshape(1, NUM_CENTERS)
    edge_feat = _tc_edge_feat(spos, dpos, centers)

    enc_bs = [b.reshape(1, -1) for b in (enc_b1, enc_b2, enc_b3, enc_b4)]
    hw, ni, no = _tc_encode(posP, dego, degi,
                            [enc_W1, enc_W2, enc_W3, enc_W4], enc_bs, g1_W)

    agg = _sc_agg(hw.reshape(NC * NP, 32), src2, dst)
    hw = _tc_mid(agg, ni, no, g1_b.reshape(1, -1), gh1_W, None)
    agg = _sc_agg(hw.reshape(NC * NP, 32), src2, dst)
    hw = _tc_mid(agg, ni, no, gh1_b.reshape(1, -1), gh2_W, jnp.tanh)
    agg = _sc_agg(hw.reshape(NC * NP, 32), src2, dst)
    hw = _tc_mid(agg, ni, no, gh2_b.reshape(1, -1), g2_W, jnp.tanh)
    agg = _sc_agg(hw.reshape(NC * NP, 32), src2, dst)

    dec_bs = [b.reshape(1, -1) for b in (dec_b1, dec_b2, dec_b3, dec_b4)]
    out = _tc_final(agg, ni, g2_b.reshape(1, -1),
                    [dec_W1, dec_W2, dec_W3, dec_W4], dec_bs)
    return out[:N], edge_feat


# speedup vs baseline: 5.6173x; 1.0326x over previous
import functools

import numpy as np
import jax
import jax.numpy as jnp
from jax import lax
from jax.experimental import pallas as pl
from jax.experimental.pallas import tpu as pltpu
from jax.experimental.pallas import tpu_sc as plsc

N = 50000
NP = 50048
E = 800000
BOX = 1.0
NUM_CENTERS = 40
GAMMA = 40.0
_CENTERS = np.linspace(0.0, 1.0, NUM_CENTERS).astype(np.float32)

NC = 2
NS = 16
CHUNK = 128
NCHUNKS = E // CHUNK

_MESH = dict(core_axis_name="c", subcore_axis_name="s", num_cores=NC,
             num_subcores=NS)

ROWS_PER_TILE = NP // NS


def _tile_chunk_range(w, nworkers):
    per = NCHUNKS // nworkers
    extra = NCHUNKS - per * nworkers
    c0 = w * per + jnp.minimum(w, extra)
    nch = per + jnp.where(w < extra, 1, 0)
    return c0, nch


def _pre_body(pos16, srcA, dstA, ones_h, zdeg_h,
              spos_out, dpos_out, dego_out, degi_out,
              sidx, didx, srows, drows, ones_v,
              degosh, degish,
              semi, semg, semo, NB=4):
    cid = lax.axis_index("c")
    sid = lax.axis_index("s")
    w = cid * NS + sid
    c0, nch = _tile_chunk_range(w, NC * NS)

    r0 = sid * ROWS_PER_TILE
    pltpu.sync_copy(zdeg_h, degosh.at[pl.ds(r0, ROWS_PER_TILE)])
    pltpu.sync_copy(zdeg_h, degish.at[pl.ds(r0, ROWS_PER_TILE)])
    pltpu.sync_copy(ones_h, ones_v)
    plsc.subcore_barrier()

    def quad(q, carry):
        del carry
        idescs = []
        for p in range(NB):
            t = q * NB + p
            tcl = jnp.minimum(t, nch - 1)
            off = (c0 + tcl) * CHUNK
            d1 = pltpu.async_copy(srcA.at[pl.ds(off, CHUNK)], sidx[p], semi[p])
            d2d = pltpu.async_copy(dstA.at[pl.ds(off, CHUNK)], didx[p], semi[p])
            idescs.append((d1, d2d, off, t))
        gdescs = []
        for p in range(NB):
            idescs[p][0].wait()
            idescs[p][1].wait()
            g1 = pltpu.async_copy(pos16.at[sidx[p]], srows[p], semg[p])
            g2 = pltpu.async_copy(pos16.at[didx[p]], drows[p], semg[p])
            gdescs.append((g1, g2))
        odescs = []
        for p in range(NB):
            gdescs[p][0].wait()
            gdescs[p][1].wait()
            _, _, off, t = idescs[p]
            odescs.append(pltpu.async_copy(
                srows[p], spos_out.at[pl.ds(off, CHUNK)], semo[p]))
            odescs.append(pltpu.async_copy(
                drows[p], dpos_out.at[pl.ds(off, CHUNK)], semo[p]))

            @pl.when(t < nch)
            def _(p=p):
                pltpu.sync_copy(ones_v, degosh.at[sidx[p]], add=True)
                pltpu.sync_copy(ones_v, degish.at[didx[p]], add=True)
        for d in odescs:
            d.wait()
        return 0

    nq = (NCHUNKS // (NC * NS) + 1 + NB - 1) // NB
    lax.fori_loop(0, nq, quad, 0)

    plsc.subcore_barrier()
    pltpu.sync_copy(degosh.at[pl.ds(r0, ROWS_PER_TILE)],
                    dego_out.at[cid, pl.ds(r0, ROWS_PER_TILE)])
    pltpu.sync_copy(degish.at[pl.ds(r0, ROWS_PER_TILE)],
                    degi_out.at[cid, pl.ds(r0, ROWS_PER_TILE)])


def _sc_pre(pos16, srcA, dstA):
    ones_h = jnp.ones((CHUNK, 16), jnp.float32)
    zdeg_h = jnp.zeros((ROWS_PER_TILE, 16), jnp.float32)
    NB = 4
    scratch = (
        [pltpu.VMEM((CHUNK,), jnp.int32) for _ in range(NB)]
        + [pltpu.VMEM((CHUNK,), jnp.int32) for _ in range(NB)]
        + [pltpu.VMEM((CHUNK, 16), jnp.float32) for _ in range(NB)]
        + [pltpu.VMEM((CHUNK, 16), jnp.float32) for _ in range(NB)]
        + [pltpu.VMEM((CHUNK, 16), jnp.float32)]
        + [pltpu.VMEM_SHARED((NP, 16), jnp.float32)]
        + [pltpu.VMEM_SHARED((NP, 16), jnp.float32)]
        + [pltpu.SemaphoreType.DMA for _ in range(NB)]
        + [pltpu.SemaphoreType.DMA for _ in range(NB)]
        + [pltpu.SemaphoreType.DMA for _ in range(NB)]
    )

    def body(pos4_r, srcA_r, dstA_r, ones_r, zdeg_r,
             spos_r, dpos_r, dego_r, degi_r, *rest):
        sidx = list(rest[0:NB])
        didx = list(rest[NB:2 * NB])
        srows = list(rest[2 * NB:3 * NB])
        drows = list(rest[3 * NB:4 * NB])
        ones_v = rest[4 * NB]
        degosh = rest[4 * NB + 1]
        degish = rest[4 * NB + 2]
        semi = list(rest[4 * NB + 3:5 * NB + 3])
        semg = list(rest[5 * NB + 3:6 * NB + 3])
        semo = list(rest[6 * NB + 3:7 * NB + 3])
        _pre_body(pos4_r, srcA_r, dstA_r, ones_r, zdeg_r,
                  spos_r, dpos_r, dego_r, degi_r,
                  sidx, didx, srows, drows, ones_v,
                  degosh, degish, semi, semg, semo, NB=NB)

    f = pl.kernel(
        body,
        out_type=(jax.ShapeDtypeStruct((E, 16), jnp.float32),
                  jax.ShapeDtypeStruct((E, 16), jnp.float32),
                  jax.ShapeDtypeStruct((NC, NP, 16), jnp.float32),
                  jax.ShapeDtypeStruct((NC, NP, 16), jnp.float32)),
        mesh=plsc.VectorSubcoreMesh(**_MESH),
        scratch_types=scratch,
        compiler_params=pltpu.CompilerParams(use_tc_tiling_on_sc=False),
    )
    return f(pos16, srcA, dstA, ones_h, zdeg_h)


def _agg_body(hw2n, src2, dstA, zrows_h,
              agg_out,
              sidx, didx, rows, aggsh, semi, semg, NB=4):
    cid = lax.axis_index("c")
    sid = lax.axis_index("s")
    c0, nch = _tile_chunk_range(sid, NS)

    r0 = sid * ROWS_PER_TILE
    pltpu.sync_copy(zrows_h, aggsh.at[pl.ds(r0, ROWS_PER_TILE)])
    plsc.subcore_barrier()

    src_base = cid * E

    def quad(q, carry):
        del carry
        idescs = []
        for p in range(NB):
            t = q * NB + p
            tcl = jnp.minimum(t, nch - 1)
            off = (c0 + tcl) * CHUNK
            d1 = pltpu.async_copy(src2.at[pl.ds(src_base + off, CHUNK)],
                                  sidx[p], semi[p])
            d2d = pltpu.async_copy(dstA.at[pl.ds(off, CHUNK)], didx[p], semi[p])
            idescs.append((d1, d2d, t))
        gdescs = []
        for p in range(NB):
            idescs[p][0].wait()
            idescs[p][1].wait()
            gdescs.append(pltpu.async_copy(hw2n.at[sidx[p]], rows[p], semg[p]))
        for p in range(NB):
            gdescs[p].wait()
            t = idescs[p][2]

            @pl.when(t < nch)
            def _(p=p):
                pltpu.sync_copy(rows[p], aggsh.at[didx[p]], add=True)
        return 0

    nq = (NCHUNKS // NS + 1 + NB - 1) // NB
    lax.fori_loop(0, nq, quad, 0)

    plsc.subcore_barrier()
    pltpu.sync_copy(aggsh.at[pl.ds(r0, ROWS_PER_TILE)],
                    agg_out.at[cid, pl.ds(r0, ROWS_PER_TILE)])


def _sc_agg(hw2n, src2, dstA):
    zrows_h = jnp.zeros((ROWS_PER_TILE, 32), jnp.float32)
    NB = 6
    scratch = (
        [pltpu.VMEM((CHUNK,), jnp.int32) for _ in range(NB)]
        + [pltpu.VMEM((CHUNK,), jnp.int32) for _ in range(NB)]
        + [pltpu.VMEM((CHUNK, 32), jnp.float32) for _ in range(NB)]
        + [pltpu.VMEM_SHARED((NP, 32), jnp.float32)]
        + [pltpu.SemaphoreType.DMA for _ in range(NB)]
        + [pltpu.SemaphoreType.DMA for _ in range(NB)]
    )

    def body(hw_r, src2_r, dstA_r, z_r, agg_r, *rest):
        sidx = list(rest[0:NB])
        didx = list(rest[NB:2 * NB])
        rows = list(rest[2 * NB:3 * NB])
        aggsh = rest[3 * NB]
        semi = list(rest[3 * NB + 1:4 * NB + 1])
        semg = list(rest[4 * NB + 1:5 * NB + 1])
        _agg_body(hw_r, src2_r, dstA_r, z_r, agg_r,
                  sidx, didx, rows, aggsh, semi, semg, NB=NB)

    f = pl.kernel(
        body,
        out_type=jax.ShapeDtypeStruct((NC, NP, 32), jnp.float32),
        mesh=plsc.VectorSubcoreMesh(**_MESH),
        scratch_types=scratch,
        compiler_params=pltpu.CompilerParams(use_tc_tiling_on_sc=False),
    )
    return f(hw2n, src2, dstA, zrows_h)


ROWB = 2176
NGRID = NP // ROWB


def _leaky(x):
    return jnp.maximum(x, 0.2 * x)


def _full_spec(shape):
    nd = len(shape)
    return pl.BlockSpec(shape, lambda i, _nd=nd: (0,) * _nd)


def _tc_encode(pos, dego, degi, Ws, bs, g1_W):
    def body(pos_r, dego_r, degi_r, w1, b1, w2, b2, w3, b3, w4, b4, g1w,
             hw_r, ni_r, no_r):
        deg_o = dego_r[0, :, 0:1] + dego_r[1, :, 0:1]
        deg_i = degi_r[0, :, 0:1] + degi_r[1, :, 0:1]
        no = lax.rsqrt(jnp.maximum(deg_o, 1.0))
        ni = lax.rsqrt(jnp.maximum(deg_i, 1.0))
        h = _leaky(pos_r[...] @ w1[...] + b1[...])
        h = _leaky(h @ w2[...] + b2[...])
        h = _leaky(h @ w3[...] + b3[...])
        h = h @ w4[...] + b4[...]
        hw = (h * no) @ g1w[...]
        hw_r[0] = hw[:, :32]
        hw_r[1] = hw[:, 32:]
        ni_r[...] = ni
        no_r[...] = no

    in_specs = [
        pl.BlockSpec((ROWB, 3), lambda i: (i, 0)),
        pl.BlockSpec((NC, ROWB, 16), lambda i: (0, i, 0)),
        pl.BlockSpec((NC, ROWB, 16), lambda i: (0, i, 0)),
    ]
    args = [pos, dego, degi]
    for W, b in zip(Ws, bs):
        in_specs += [_full_spec(W.shape), _full_spec(b.shape)]
        args += [W, b]
    in_specs.append(_full_spec(g1_W.shape))
    args.append(g1_W)
    out_specs = (
        pl.BlockSpec((NC, ROWB, 32), lambda i: (0, i, 0)),
        pl.BlockSpec((ROWB, 1), lambda i: (i, 0)),
        pl.BlockSpec((ROWB, 1), lambda i: (i, 0)),
    )
    return pl.pallas_call(
        body,
        grid=(NGRID,),
        in_specs=in_specs,
        out_specs=out_specs,
        out_shape=(jax.ShapeDtypeStruct((NC, NP, 32), jnp.float32),
                   jax.ShapeDtypeStruct((NP, 1), jnp.float32),
                   jax.ShapeDtypeStruct((NP, 1), jnp.float32)),
    )(*args)


def _tc_mid(aggp, ni, no, b_prev, W_next, act):
    def body(agg_r, ni_r, no_r, b_r, w_r, hw_r):
        agg = jnp.concatenate([agg_r[0], agg_r[1]], axis=1)
        h = agg * ni_r[...] + b_r[...]
        if act is not None:
            h = act(h)
        hw = (h * no_r[...]) @ w_r[...]
        hw_r[0] = hw[:, :32]
        hw_r[1] = hw[:, 32:]

    return pl.pallas_call(
        body,
        grid=(NGRID,),
        in_specs=[
            pl.BlockSpec((NC, ROWB, 32), lambda i: (0, i, 0)),
            pl.BlockSpec((ROWB, 1), lambda i: (i, 0)),
            pl.BlockSpec((ROWB, 1), lambda i: (i, 0)),
            _full_spec(b_prev.shape),
            _full_spec(W_next.shape),
        ],
        out_specs=pl.BlockSpec((NC, ROWB, 32), lambda i: (0, i, 0)),
        out_shape=jax.ShapeDtypeStruct((NC, NP, 32), jnp.float32),
    )(aggp, ni, no, b_prev, W_next)


def _tc_final(aggp, ni, b_prev, Ws, bs):
    def body(agg_r, ni_r, b_r, w1, b1, w2, b2, w3, b3, w4, b4, out_r):
        agg = jnp.concatenate([agg_r[0], agg_r[1]], axis=1)
        h = agg * ni_r[...] + b_r[...]
        h = _leaky(h @ w1[...] + b1[...])
        h = _leaky(h @ w2[...] + b2[...])
        h = _leaky(h @ w3[...] + b3[...])
        out_r[...] = h @ w4[...] + b4[...]

    in_specs = [
        pl.BlockSpec((NC, ROWB, 32), lambda i: (0, i, 0)),
        pl.BlockSpec((ROWB, 1), lambda i: (i, 0)),
        _full_spec(b_prev.shape),
    ]
    args = [aggp, ni, b_prev]
    for W, b in zip(Ws, bs):
        in_specs += [_full_spec(W.shape), _full_spec(b.shape)]
        args += [W, b]
    return pl.pallas_call(
        body,
        grid=(NGRID,),
        in_specs=in_specs,
        out_specs=pl.BlockSpec((ROWB, 3), lambda i: (i, 0)),
        out_shape=jax.ShapeDtypeStruct((NP, 3), jnp.float32),
    )(*args)


EBLK = 4000
EGRID = E // EBLK


def _tc_edge_feat(spos, dpos, centers):
    def body(s_r, d_r, c_r, ef_r):
        sp = s_r[...]
        dp = d_r[...]
        d2 = None
        for comp in range(3):
            r = (dp[:, comp:comp + 1] - sp[:, comp:comp + 1]) + 0.5
            fl = jnp.floor(r)
            wv = r - fl - 0.5
            d2 = wv * wv if d2 is None else d2 + wv * wv
        dist = jnp.sqrt(d2)
        ef_r[...] = jnp.exp(-GAMMA * (dist - c_r[...]) ** 2)

    return pl.pallas_call(
        body,
        grid=(EGRID,),
        in_specs=[
            pl.BlockSpec((EBLK, 16), lambda i: (i, 0)),
            pl.BlockSpec((EBLK, 16), lambda i: (i, 0)),
            _full_spec((1, NUM_CENTERS)),
        ],
        out_specs=pl.BlockSpec((EBLK, NUM_CENTERS), lambda i: (i, 0)),
        out_shape=jax.ShapeDtypeStruct((E, NUM_CENTERS), jnp.float32),
    )(spos, dpos, centers)


def kernel(pos, edge_index, enc_W1, enc_b1, enc_W2, enc_b2, enc_W3, enc_b3,
           enc_W4, enc_b4, g1_W, g1_b, gh1_W, gh1_b, gh2_W, gh2_b, g2_W, g2_b,
           dec_W1, dec_b1, dec_W2, dec_b2, dec_W3, dec_b3, dec_W4, dec_b4):
    src = edge_index[0]
    dst = edge_index[1]
    pos16 = jnp.pad(pos, ((0, NP - N), (0, 13)))
    posP = jnp.pad(pos, ((0, NP - N), (0, 0)))
    src2 = jnp.concatenate([src, src + NP])

    spos, dpos, dego, degi = _sc_pre(pos16, src, dst)

    centers = jnp.asarray(_CENTERS).reshape(1, NUM_CENTERS)
    edge_feat = _tc_edge_feat(spos, dpos, centers)

    enc_bs = [b.reshape(1, -1) for b in (enc_b1, enc_b2, enc_b3, enc_b4)]
    hw, ni, no = _tc_encode(posP, dego, degi,
                            [enc_W1, enc_W2, enc_W3, enc_W4], enc_bs, g1_W)

    agg = _sc_agg(hw.reshape(NC * NP, 32), src2, dst)
    hw = _tc_mid(agg, ni, no, g1_b.reshape(1, -1), gh1_W, None)
    agg = _sc_agg(hw.reshape(NC * NP, 32), src2, dst)
    hw = _tc_mid(agg, ni, no, gh1_b.reshape(1, -1), gh2_W, jnp.tanh)
    agg = _sc_agg(hw.reshape(NC * NP, 32), src2, dst)
    hw = _tc_mid(agg, ni, no, gh2_b.reshape(1, -1), g2_W, jnp.tanh)
    agg = _sc_agg(hw.reshape(NC * NP, 32), src2, dst)

    dec_bs = [b.reshape(1, -1) for b in (dec_b1, dec_b2, dec_b3, dec_b4)]
    out = _tc_final(agg, ni, g2_b.reshape(1, -1),
                    [dec_W1, dec_W2, dec_W3, dec_W4], dec_bs)
    return out[:N], edge_feat


# speedup vs baseline: 7.5667x; 1.3470x over previous
import functools

import numpy as np
import jax
import jax.numpy as jnp
from jax import lax
from jax.experimental import pallas as pl
from jax.experimental.pallas import tpu as pltpu
from jax.experimental.pallas import tpu_sc as plsc

N = 50000
NP = 50048
E = 800000
BOX = 1.0
NUM_CENTERS = 40
GAMMA = 40.0
_CENTERS = np.linspace(0.0, 1.0, NUM_CENTERS).astype(np.float32)

NC = 2
NS = 16
CHUNK = 128
NCHUNKS = E // CHUNK
NCHUNKS_PAD = 6272

_MESH = dict(core_axis_name="c", subcore_axis_name="s", num_cores=NC,
             num_subcores=NS)

ROWS_PER_TILE = NP // NS


def _tile_chunk_range(w, nworkers):
    per = NCHUNKS // nworkers
    extra = NCHUNKS - per * nworkers
    c0 = w * per + jnp.minimum(w, extra)
    nch = per + jnp.where(w < extra, 1, 0)
    return c0, nch


def _pre_body(pos16, srcA, dstA, ones_h, zdeg_h,
              d2m_out, dego_out, degi_out,
              sidx, didx, srows, drows, d2buf, ones_v,
              degosh, degish,
              semi, semg, semo, NB=4):
    cid = lax.axis_index("c")
    sid = lax.axis_index("s")
    w = cid * NS + sid
    c0, nch = _tile_chunk_range(w, NC * NS)

    r0 = sid * ROWS_PER_TILE
    pltpu.sync_copy(zdeg_h, degosh.at[pl.ds(r0, ROWS_PER_TILE)])
    pltpu.sync_copy(zdeg_h, degish.at[pl.ds(r0, ROWS_PER_TILE)])
    pltpu.sync_copy(ones_h, ones_v)
    plsc.subcore_barrier()

    lane = lax.iota(jnp.int32, 16)
    cols = [jnp.full((16,), c, jnp.int32) for c in range(3)]

    def quad(q, carry):
        del carry
        idescs = []
        for p in range(NB):
            t = q * NB + p
            tcl = jnp.minimum(t, nch - 1)
            off = (c0 + tcl) * CHUNK
            d1 = pltpu.async_copy(srcA.at[pl.ds(off, CHUNK)], sidx[p], semi[p])
            d2d = pltpu.async_copy(dstA.at[pl.ds(off, CHUNK)], didx[p], semi[p])
            idescs.append((d1, d2d, c0 + tcl, t))
        gdescs = []
        for p in range(NB):
            idescs[p][0].wait()
            idescs[p][1].wait()
            g1 = pltpu.async_copy(pos16.at[sidx[p]], srows[p], semg[p])
            g2 = pltpu.async_copy(pos16.at[didx[p]], drows[p], semg[p])
            gdescs.append((g1, g2))
        odescs = []
        for p in range(NB):
            gdescs[p][0].wait()
            gdescs[p][1].wait()
            _, _, g, t = idescs[p]
            for j in range(CHUNK // 16):
                ridx = lane + (j * 16)
                acc = None
                for comp in range(3):
                    sv = plsc.load_gather(srows[p], [ridx, cols[comp]])
                    dv = plsc.load_gather(drows[p], [ridx, cols[comp]])
                    r = (dv - sv) + 0.5
                    ti = r.astype(jnp.int32)
                    tf = ti.astype(jnp.float32)
                    fl = jnp.where(r < tf, tf - 1.0, tf)
                    wv = r - fl - 0.5
                    acc = wv * wv if acc is None else acc + wv * wv
                d2buf[p][pl.ds(j * 16, 16)] = acc
            odescs.append(pltpu.async_copy(d2buf[p], d2m_out.at[g], semo[p]))

            @pl.when(t < nch)
            def _(p=p):
                pltpu.sync_copy(ones_v, degosh.at[sidx[p]], add=True)
                pltpu.sync_copy(ones_v, degish.at[didx[p]], add=True)
        for d in odescs:
            d.wait()
        return 0

    nq = (NCHUNKS // (NC * NS) + 1 + NB - 1) // NB
    lax.fori_loop(0, nq, quad, 0)

    plsc.subcore_barrier()
    pltpu.sync_copy(degosh.at[pl.ds(r0, ROWS_PER_TILE)],
                    dego_out.at[cid, pl.ds(r0, ROWS_PER_TILE)])
    pltpu.sync_copy(degish.at[pl.ds(r0, ROWS_PER_TILE)],
                    degi_out.at[cid, pl.ds(r0, ROWS_PER_TILE)])


def _sc_pre(pos16, srcA, dstA):
    ones_h = jnp.ones((CHUNK, 16), jnp.float32)
    zdeg_h = jnp.zeros((ROWS_PER_TILE, 16), jnp.float32)
    NB = 4
    scratch = (
        [pltpu.VMEM((CHUNK,), jnp.int32) for _ in range(NB)]
        + [pltpu.VMEM((CHUNK,), jnp.int32) for _ in range(NB)]
        + [pltpu.VMEM((CHUNK, 16), jnp.float32) for _ in range(NB)]
        + [pltpu.VMEM((CHUNK, 16), jnp.float32) for _ in range(NB)]
        + [pltpu.VMEM((CHUNK,), jnp.float32) for _ in range(NB)]
        + [pltpu.VMEM((CHUNK, 16), jnp.float32)]
        + [pltpu.VMEM_SHARED((NP, 16), jnp.float32)]
        + [pltpu.VMEM_SHARED((NP, 16), jnp.float32)]
        + [pltpu.SemaphoreType.DMA for _ in range(NB)]
        + [pltpu.SemaphoreType.DMA for _ in range(NB)]
        + [pltpu.SemaphoreType.DMA for _ in range(NB)]
    )

    def body(pos16_r, srcA_r, dstA_r, ones_r, zdeg_r,
             d2m_r, dego_r, degi_r, *rest):
        sidx = list(rest[0:NB])
        didx = list(rest[NB:2 * NB])
        srows = list(rest[2 * NB:3 * NB])
        drows = list(rest[3 * NB:4 * NB])
        d2buf = list(rest[4 * NB:5 * NB])
        ones_v = rest[5 * NB]
        degosh = rest[5 * NB + 1]
        degish = rest[5 * NB + 2]
        semi = list(rest[5 * NB + 3:6 * NB + 3])
        semg = list(rest[6 * NB + 3:7 * NB + 3])
        semo = list(rest[7 * NB + 3:8 * NB + 3])
        _pre_body(pos16_r, srcA_r, dstA_r, ones_r, zdeg_r,
                  d2m_r, dego_r, degi_r,
                  sidx, didx, srows, drows, d2buf, ones_v,
                  degosh, degish, semi, semg, semo, NB=NB)

    f = pl.kernel(
        body,
        out_type=(jax.ShapeDtypeStruct((NCHUNKS_PAD, CHUNK), jnp.float32),
                  jax.ShapeDtypeStruct((NC, NP, 16), jnp.float32),
                  jax.ShapeDtypeStruct((NC, NP, 16), jnp.float32)),
        mesh=plsc.VectorSubcoreMesh(**_MESH),
        scratch_types=scratch,
        compiler_params=pltpu.CompilerParams(use_tc_tiling_on_sc=False,
                                             needs_layout_passes=False),
    )
    return f(pos16, srcA, dstA, ones_h, zdeg_h)


def _agg_body(hw2n, src2, dstA, zrows_h,
              agg_out,
              sidx, didx, rows, aggsh, semi, semg, NB=4):
    cid = lax.axis_index("c")
    sid = lax.axis_index("s")
    c0, nch = _tile_chunk_range(sid, NS)

    r0 = sid * ROWS_PER_TILE
    pltpu.sync_copy(zrows_h, aggsh.at[pl.ds(r0, ROWS_PER_TILE)])
    plsc.subcore_barrier()

    src_base = cid * E

    def quad(q, carry):
        del carry
        idescs = []
        for p in range(NB):
            t = q * NB + p
            tcl = jnp.minimum(t, nch - 1)
            off = (c0 + tcl) * CHUNK
            d1 = pltpu.async_copy(src2.at[pl.ds(src_base + off, CHUNK)],
                                  sidx[p], semi[p])
            d2d = pltpu.async_copy(dstA.at[pl.ds(off, CHUNK)], didx[p], semi[p])
            idescs.append((d1, d2d, t))
        gdescs = []
        for p in range(NB):
            idescs[p][0].wait()
            idescs[p][1].wait()
            gdescs.append(pltpu.async_copy(hw2n.at[sidx[p]], rows[p], semg[p]))
        for p in range(NB):
            gdescs[p].wait()
            t = idescs[p][2]

            @pl.when(t < nch)
            def _(p=p):
                pltpu.sync_copy(rows[p], aggsh.at[didx[p]], add=True)
        return 0

    nq = (NCHUNKS // NS + 1 + NB - 1) // NB
    lax.fori_loop(0, nq, quad, 0)

    plsc.subcore_barrier()
    pltpu.sync_copy(aggsh.at[pl.ds(r0, ROWS_PER_TILE)],
                    agg_out.at[cid, pl.ds(r0, ROWS_PER_TILE)])


def _sc_agg(hw2n, src2, dstA):
    zrows_h = jnp.zeros((ROWS_PER_TILE, 32), jnp.float32)
    NB = 6
    scratch = (
        [pltpu.VMEM((CHUNK,), jnp.int32) for _ in range(NB)]
        + [pltpu.VMEM((CHUNK,), jnp.int32) for _ in range(NB)]
        + [pltpu.VMEM((CHUNK, 32), jnp.float32) for _ in range(NB)]
        + [pltpu.VMEM_SHARED((NP, 32), jnp.float32)]
        + [pltpu.SemaphoreType.DMA for _ in range(NB)]
        + [pltpu.SemaphoreType.DMA for _ in range(NB)]
    )

    def body(hw_r, src2_r, dstA_r, z_r, agg_r, *rest):
        sidx = list(rest[0:NB])
        didx = list(rest[NB:2 * NB])
        rows = list(rest[2 * NB:3 * NB])
        aggsh = rest[3 * NB]
        semi = list(rest[3 * NB + 1:4 * NB + 1])
        semg = list(rest[4 * NB + 1:5 * NB + 1])
        _agg_body(hw_r, src2_r, dstA_r, z_r, agg_r,
                  sidx, didx, rows, aggsh, semi, semg, NB=NB)

    f = pl.kernel(
        body,
        out_type=jax.ShapeDtypeStruct((NC, NP, 32), jnp.float32),
        mesh=plsc.VectorSubcoreMesh(**_MESH),
        scratch_types=scratch,
        compiler_params=pltpu.CompilerParams(use_tc_tiling_on_sc=False),
    )
    return f(hw2n, src2, dstA, zrows_h)


ROWB = 2176
NGRID = NP // ROWB


def _leaky(x):
    return jnp.maximum(x, 0.2 * x)


def _full_spec(shape):
    nd = len(shape)
    return pl.BlockSpec(shape, lambda i, _nd=nd: (0,) * _nd)


def _tc_encode(pos, dego, degi, Ws, bs, g1_W):
    def body(pos_r, dego_r, degi_r, w1, b1, w2, b2, w3, b3, w4, b4, g1w,
             hw_r, ni_r, no_r):
        deg_o = dego_r[0, :, 0:1] + dego_r[1, :, 0:1]
        deg_i = degi_r[0, :, 0:1] + degi_r[1, :, 0:1]
        no = lax.rsqrt(jnp.maximum(deg_o, 1.0))
        ni = lax.rsqrt(jnp.maximum(deg_i, 1.0))
        h = _leaky(pos_r[...] @ w1[...] + b1[...])
        h = _leaky(h @ w2[...] + b2[...])
        h = _leaky(h @ w3[...] + b3[...])
        h = h @ w4[...] + b4[...]
        hw = (h * no) @ g1w[...]
        hw_r[0] = hw[:, :32]
        hw_r[1] = hw[:, 32:]
        ni_r[...] = ni
        no_r[...] = no

    in_specs = [
        pl.BlockSpec((ROWB, 3), lambda i: (i, 0)),
        pl.BlockSpec((NC, ROWB, 16), lambda i: (0, i, 0)),
        pl.BlockSpec((NC, ROWB, 16), lambda i: (0, i, 0)),
    ]
    args = [pos, dego, degi]
    for W, b in zip(Ws, bs):
        in_specs += [_full_spec(W.shape), _full_spec(b.shape)]
        args += [W, b]
    in_specs.append(_full_spec(g1_W.shape))
    args.append(g1_W)
    out_specs = (
        pl.BlockSpec((NC, ROWB, 32), lambda i: (0, i, 0)),
        pl.BlockSpec((ROWB, 1), lambda i: (i, 0)),
        pl.BlockSpec((ROWB, 1), lambda i: (i, 0)),
    )
    return pl.pallas_call(
        body,
        grid=(NGRID,),
        in_specs=in_specs,
        out_specs=out_specs,
        out_shape=(jax.ShapeDtypeStruct((NC, NP, 32), jnp.float32),
                   jax.ShapeDtypeStruct((NP, 1), jnp.float32),
                   jax.ShapeDtypeStruct((NP, 1), jnp.float32)),
    )(*args)


def _tc_mid(aggp, ni, no, b_prev, W_next, act):
    def body(agg_r, ni_r, no_r, b_r, w_r, hw_r):
        agg = jnp.concatenate([agg_r[0], agg_r[1]], axis=1)
        h = agg * ni_r[...] + b_r[...]
        if act is not None:
            h = act(h)
        hw = (h * no_r[...]) @ w_r[...]
        hw_r[0] = hw[:, :32]
        hw_r[1] = hw[:, 32:]

    return pl.pallas_call(
        body,
        grid=(NGRID,),
        in_specs=[
            pl.BlockSpec((NC, ROWB, 32), lambda i: (0, i, 0)),
            pl.BlockSpec((ROWB, 1), lambda i: (i, 0)),
            pl.BlockSpec((ROWB, 1), lambda i: (i, 0)),
            _full_spec(b_prev.shape),
            _full_spec(W_next.shape),
        ],
        out_specs=pl.BlockSpec((NC, ROWB, 32), lambda i: (0, i, 0)),
        out_shape=jax.ShapeDtypeStruct((NC, NP, 32), jnp.float32),
    )(aggp, ni, no, b_prev, W_next)


def _tc_final(aggp, ni, b_prev, Ws, bs):
    def body(agg_r, ni_r, b_r, w1, b1, w2, b2, w3, b3, w4, b4, out_r):
        agg = jnp.concatenate([agg_r[0], agg_r[1]], axis=1)
        h = agg * ni_r[...] + b_r[...]
        h = _leaky(h @ w1[...] + b1[...])
        h = _leaky(h @ w2[...] + b2[...])
        h = _leaky(h @ w3[...] + b3[...])
        out_r[...] = h @ w4[...] + b4[...]

    in_specs = [
        pl.BlockSpec((NC, ROWB, 32), lambda i: (0, i, 0)),
        pl.BlockSpec((ROWB, 1), lambda i: (i, 0)),
        _full_spec(b_prev.shape),
    ]
    args = [aggp, ni, b_prev]
    for W, b in zip(Ws, bs):
        in_specs += [_full_spec(W.shape), _full_spec(b.shape)]
        args += [W, b]
    return pl.pallas_call(
        body,
        grid=(NGRID,),
        in_specs=in_specs,
        out_specs=pl.BlockSpec((ROWB, 3), lambda i: (i, 0)),
        out_shape=jax.ShapeDtypeStruct((NP, 3), jnp.float32),
    )(*args)


EBLK = 4000
EGRID = E // EBLK


EF_CB = 128


def _tc_edge_feat(d2t, centers):
    def body(d2_r, c_r, ef_r):
        for c in range(EF_CB):
            dist = jnp.sqrt(d2_r[:, c:c + 1])
            ef_r[pl.ds(c * CHUNK, CHUNK), :] = (
                jnp.exp(-GAMMA * (dist - c_r[...]) ** 2))

    return pl.pallas_call(
        body,
        grid=(NCHUNKS_PAD // EF_CB,),
        in_specs=[
            pl.BlockSpec((CHUNK, EF_CB), lambda i: (0, i)),
            _full_spec((1, NUM_CENTERS)),
        ],
        out_specs=pl.BlockSpec((CHUNK * EF_CB, NUM_CENTERS), lambda i: (i, 0)),
        out_shape=jax.ShapeDtypeStruct((E, NUM_CENTERS), jnp.float32),
    )(d2t, centers)


def kernel(pos, edge_index, enc_W1, enc_b1, enc_W2, enc_b2, enc_W3, enc_b3,
           enc_W4, enc_b4, g1_W, g1_b, gh1_W, gh1_b, gh2_W, gh2_b, g2_W, g2_b,
           dec_W1, dec_b1, dec_W2, dec_b2, dec_W3, dec_b3, dec_W4, dec_b4):
    src = edge_index[0]
    dst = edge_index[1]
    pos16 = jnp.pad(pos, ((0, NP - N), (0, 13)))
    posP = jnp.pad(pos, ((0, NP - N), (0, 0)))
    src2 = jnp.concatenate([src, src + NP])

    d2m, dego, degi = _sc_pre(pos16, src, dst)

    centers = jnp.asarray(_CENTERS).reshape(1, NUM_CENTERS)
    edge_feat = _tc_edge_feat(d2m.T, centers)

    enc_bs = [b.reshape(1, -1) for b in (enc_b1, enc_b2, enc_b3, enc_b4)]
    hw, ni, no = _tc_encode(posP, dego, degi,
                            [enc_W1, enc_W2, enc_W3, enc_W4], enc_bs, g1_W)

    agg = _sc_agg(hw.reshape(NC * NP, 32), src2, dst)
    hw = _tc_mid(agg, ni, no, g1_b.reshape(1, -1), gh1_W, None)
    agg = _sc_agg(hw.reshape(NC * NP, 32), src2, dst)
    hw = _tc_mid(agg, ni, no, gh1_b.reshape(1, -1), gh2_W, jnp.tanh)
    agg = _sc_agg(hw.reshape(NC * NP, 32), src2, dst)
    hw = _tc_mid(agg, ni, no, gh2_b.reshape(1, -1), g2_W, jnp.tanh)
    agg = _sc_agg(hw.reshape(NC * NP, 32), src2, dst)

    dec_bs = [b.reshape(1, -1) for b in (dec_b1, dec_b2, dec_b3, dec_b4)]
    out = _tc_final(agg, ni, g2_b.reshape(1, -1),
                    [dec_W1, dec_W2, dec_W3, dec_W4], dec_bs)
    return out[:N], edge_feat
